# baseline (speedup 1.0000x reference)
import functools

import jax
import jax.numpy as jnp
from jax import lax
from jax.experimental import pallas as pl
from jax.experimental.pallas import tpu as pltpu
from jax.experimental.pallas import tpu_sc as plsc

N = 50000
E = 800000
IN_DIM = 25
EDGE_DIM = 11
H = 64
DEPTH = 4
NUM_GRAPHS = 1000
EPS = 1e-5

NC = 2
NS = 16
L = 16

PASSES = 2
QUART = 12544
VROWS = 6400
DUMMY = 6300
C = 80
EPT = E // NS
NCHUNK = EPT // C
ZPT = VROWS // NS

GPR = 512
PC = 80
NODE_CHUNKS = N // PC
PITER = (NODE_CHUNKS + NC * NS - 1) // (NC * NS)

_SC_MESH = plsc.VectorSubcoreMesh(
    core_axis_name="c", subcore_axis_name="s", num_cores=NC, num_subcores=NS)


def _leaky(t):
    return jnp.maximum(t, 0.1 * t)


def _edge_body(hw_hbm, src_hbm, dst_hbm, ep_hbm, out_hbm,
               src_v, dst_v, idx_v, rows_v, ep_v, sel_v, aggr_sh, sem):
    c = lax.axis_index("c")
    s = lax.axis_index("s")
    zerov = jnp.zeros((L,), jnp.float32)

    for p in range(PASSES):
        q = c * PASSES + p
        base_node = q * QUART

        def _zfill(i, _):
            for k in range(2 * H // L):
                sel_v[i, pl.ds(k * L, L)] = zerov
            return 0
        lax.fori_loop(0, C, _zfill, 0)
        zbase = s * ZPT
        for z in range(ZPT // C):
            pltpu.sync_copy(sel_v, aggr_sh.at[pl.ds(zbase + z * C, C)])
        plsc.subcore_barrier()

        def _chunk(k, _):
            ebase = s * EPT + k * C
            pltpu.sync_copy(src_hbm.at[pl.ds(ebase, C)], src_v)
            pltpu.sync_copy(dst_hbm.at[pl.ds(ebase, C)], dst_v.at[pl.ds(0, C)])
            pltpu.sync_copy(ep_hbm.at[pl.ds(ebase, C)], ep_v)
            pltpu.async_copy(hw_hbm.at[src_v], rows_v, sem).wait()
            for t in range(C // L):
                d = dst_v[pl.ds(t * L, L)]
                loc = d - base_node
                ok = (loc >= 0) & (loc < QUART)
                idx_v[pl.ds(t * L, L)] = jnp.where(
                    ok, lax.shift_right_logical(loc, 1), DUMMY)

            def _msg(e, _):
                d16 = dst_v[pl.ds(e, L)]
                pe = lax.rem(d16[0], 2)
                mlow = pe == 0
                mhigh = pe == 1
                for k2 in range(H // L):
                    a = rows_v[e, pl.ds(k2 * L, L)]
                    b = ep_v[e, pl.ds(k2 * L, L)]
                    v = a + b
                    m = jnp.maximum(v, 0.1 * v)
                    sel_v[e, pl.ds(k2 * L, L)] = jnp.where(mlow, m, zerov)
                    sel_v[e, pl.ds(H + k2 * L, L)] = jnp.where(mhigh, m,
                                                               zerov)
                return 0
            lax.fori_loop(0, C, _msg, 0)
            pltpu.sync_copy(sel_v, aggr_sh.at[idx_v], add=True)
            return 0
        lax.fori_loop(0, NCHUNK, _chunk, 0)

        plsc.subcore_barrier()
        pltpu.sync_copy(aggr_sh.at[pl.ds(s * ZPT, ZPT)],
                        out_hbm.at[q, pl.ds(s * ZPT, ZPT)])
        plsc.subcore_barrier()


_edge_call = pl.kernel(
    _edge_body,
    out_type=jax.ShapeDtypeStruct((NC * PASSES, VROWS, 2 * H), jnp.float32),
    mesh=_SC_MESH,
    scratch_types=[
        pltpu.VMEM((C,), jnp.int32),
        pltpu.VMEM((C + L,), jnp.int32),
        pltpu.VMEM((C,), jnp.int32),
        pltpu.VMEM((C, 2 * H), jnp.float32),
        pltpu.VMEM((C, H), jnp.float32),
        pltpu.VMEM((C, 2 * H), jnp.float32),
        pltpu.VMEM_SHARED((VROWS, 2 * H), jnp.float32),
        pltpu.SemaphoreType.DMA,
    ],
)


def _pool_body(h_hbm, batch_hbm, sums_hbm, cnt_hbm,
               bidx_v, idx_v, h_v, hsel_v, csel_v, sums_sh, cnt_sh, sem):
    c = lax.axis_index("c")
    s = lax.axis_index("s")
    wid = s * NC + c
    zerov = jnp.zeros((L,), jnp.float32)
    lane = lax.broadcasted_iota(jnp.int32, (L,), 0)
    onerow = jnp.where(lane == 0, 1.0, 0.0).astype(jnp.float32)

    def _zf(i, _):
        for k in range(2 * H // L):
            hsel_v[i, pl.ds(k * L, L)] = zerov
            csel_v[i, pl.ds(k * L, L)] = zerov
        return 0
    lax.fori_loop(0, PC, _zf, 0)

    zpt = GPR // NS
    pltpu.sync_copy(hsel_v.at[pl.ds(0, zpt)], sums_sh.at[pl.ds(s * zpt, zpt)])
    pltpu.sync_copy(csel_v.at[pl.ds(0, zpt)], cnt_sh.at[pl.ds(s * zpt, zpt)])
    plsc.subcore_barrier()

    for i in range(PITER):
        k = wid + NC * NS * i

        @pl.when(k < NODE_CHUNKS)
        def _do(k=k):
            nbase = k * PC
            pltpu.sync_copy(batch_hbm.at[pl.ds(nbase, PC)],
                            bidx_v.at[pl.ds(0, PC)])
            pltpu.sync_copy(h_hbm.at[pl.ds(nbase, PC)], h_v)
            for t in range(PC // L):
                g = bidx_v[pl.ds(t * L, L)]
                idx_v[pl.ds(t * L, L)] = lax.shift_right_logical(g, 1)

            def _sel(e, _):
                g16 = bidx_v[pl.ds(e, L)]
                pe = lax.rem(g16[0], 2)
                for k2 in range(H // L):
                    m = h_v[e, pl.ds(k2 * L, L)]
                    hsel_v[e, pl.ds(k2 * L, L)] = jnp.where(pe == 0, m, zerov)
                    hsel_v[e, pl.ds(H + k2 * L, L)] = jnp.where(pe == 1, m,
                                                                zerov)
                csel_v[e, pl.ds(0, L)] = jnp.where(pe == 0, onerow, zerov)
                csel_v[e, pl.ds(H, L)] = jnp.where(pe == 1, onerow, zerov)
                return 0
            lax.fori_loop(0, PC, _sel, 0)
            pltpu.sync_copy(hsel_v, sums_sh.at[idx_v], add=True)
            pltpu.sync_copy(csel_v, cnt_sh.at[idx_v], add=True)

    plsc.subcore_barrier()
    pltpu.sync_copy(sums_sh.at[pl.ds(s * zpt, zpt)],
                    sums_hbm.at[c, pl.ds(s * zpt, zpt)])
    pltpu.sync_copy(cnt_sh.at[pl.ds(s * zpt, zpt)],
                    cnt_hbm.at[c, pl.ds(s * zpt, zpt)])


_pool_call = pl.kernel(
    _pool_body,
    out_type=(jax.ShapeDtypeStruct((NC, GPR, 2 * H), jnp.float32),
              jax.ShapeDtypeStruct((NC, GPR, 2 * H), jnp.float32)),
    mesh=_SC_MESH,
    scratch_types=[
        pltpu.VMEM((PC + L,), jnp.int32),
        pltpu.VMEM((PC,), jnp.int32),
        pltpu.VMEM((PC, H), jnp.float32),
        pltpu.VMEM((PC, 2 * H), jnp.float32),
        pltpu.VMEM((PC, 2 * H), jnp.float32),
        pltpu.VMEM_SHARED((GPR, 2 * H), jnp.float32),
        pltpu.VMEM_SHARED((GPR, 2 * H), jnp.float32),
        pltpu.SemaphoreType.DMA,
    ],
)


BN = 2000
BE = 4000


def _node_in_body(x_ref, w_ref, b_ref, wt_ref, h_ref, hw_ref):
    h = _leaky(jnp.dot(x_ref[...], w_ref[...],
                       preferred_element_type=jnp.float32) + b_ref[...])
    h_ref[...] = h
    hw_ref[...] = jnp.dot(h, wt_ref[...], preferred_element_type=jnp.float32)


def _node_in(x_pad, w_pad, b, wt):
    return pl.pallas_call(
        _node_in_body,
        grid=(N // BN,),
        in_specs=[pl.BlockSpec((BN, 32), lambda i: (i, 0)),
                  pl.BlockSpec((32, H), lambda i: (0, 0)),
                  pl.BlockSpec((1, H), lambda i: (0, 0)),
                  pl.BlockSpec((H, 2 * H), lambda i: (0, 0))],
        out_specs=[pl.BlockSpec((BN, H), lambda i: (i, 0)),
                   pl.BlockSpec((BN, 2 * H), lambda i: (i, 0))],
        out_shape=[jax.ShapeDtypeStruct((N, H), jnp.float32),
                   jax.ShapeDtypeStruct((N, 2 * H), jnp.float32)],
    )(x_pad, w_pad, b, wt)


def _eproj_body(ea_ref, wb_ref, bias_ref, o0, o1, o2, o3):
    t = jnp.dot(ea_ref[...], wb_ref[...],
                preferred_element_type=jnp.float32) + bias_ref[...]
    o0[...] = t[:, 0:H]
    o1[...] = t[:, H:2 * H]
    o2[...] = t[:, 2 * H:3 * H]
    o3[...] = t[:, 3 * H:4 * H]


def _eproj(ea_pad, wb_all, b_all):
    return pl.pallas_call(
        _eproj_body,
        grid=(E // BE,),
        in_specs=[pl.BlockSpec((BE, 16), lambda i: (i, 0)),
                  pl.BlockSpec((16, 4 * H), lambda i: (0, 0)),
                  pl.BlockSpec((1, 4 * H), lambda i: (0, 0))],
        out_specs=[pl.BlockSpec((BE, H), lambda i: (i, 0))] * 4,
        out_shape=[jax.ShapeDtypeStruct((E, H), jnp.float32)] * 4,
    )(ea_pad, wb_all, b_all)


def _update_body(a_ref, h_ref, wua_ref, wuh_ref, ub_ref, g_ref, bb_ref,
                 wn_ref, hn_ref, hwn_ref):
    z = (jnp.dot(a_ref[...], wua_ref[...], preferred_element_type=jnp.float32)
         + jnp.dot(h_ref[...], wuh_ref[...], preferred_element_type=jnp.float32)
         + ub_ref[...])
    r = jnp.maximum(z, 0.0)
    hn = g_ref[...] * r + bb_ref[...]
    hn_ref[...] = hn
    hwn_ref[...] = jnp.dot(hn, wn_ref[...], preferred_element_type=jnp.float32)


def _update(aggr, h, wua, wuh, ub, g, bb, wn):
    return pl.pallas_call(
        _update_body,
        grid=(N // BN,),
        in_specs=[pl.BlockSpec((BN, H), lambda i: (i, 0)),
                  pl.BlockSpec((BN, H), lambda i: (i, 0)),
                  pl.BlockSpec((H, H), lambda i: (0, 0)),
                  pl.BlockSpec((H, H), lambda i: (0, 0)),
                  pl.BlockSpec((1, H), lambda i: (0, 0)),
                  pl.BlockSpec((1, H), lambda i: (0, 0)),
                  pl.BlockSpec((1, H), lambda i: (0, 0)),
                  pl.BlockSpec((H, 2 * H), lambda i: (0, 0))],
        out_specs=[pl.BlockSpec((BN, H), lambda i: (i, 0)),
                   pl.BlockSpec((BN, 2 * H), lambda i: (i, 0))],
        out_shape=[jax.ShapeDtypeStruct((N, H), jnp.float32),
                   jax.ShapeDtypeStruct((N, 2 * H), jnp.float32)],
    )(aggr, h, wua, wuh, ub, g, bb, wn)


def _combine_body(s_ref, c_ref, lo_ref, hi_ref):
    sm = s_ref[0] + s_ref[1]
    ct = c_ref[0] + c_ref[1]
    lo_ref[...] = sm[:, 0:H] / jnp.maximum(ct[:, 0:1], 1.0)
    hi_ref[...] = sm[:, H:2 * H] / jnp.maximum(ct[:, H:H + 1], 1.0)


def _combine(sums2, cnt2):
    return pl.pallas_call(
        _combine_body,
        out_shape=[jax.ShapeDtypeStruct((GPR, H), jnp.float32),
                   jax.ShapeDtypeStruct((GPR, H), jnp.float32)],
    )(sums2, cnt2)


def kernel(x, edge_index, edge_attr, batch, in_W, in_b,
           msg_W_0, msg_b_0, up_W_0, up_b_0, bn_g_0, bn_b_0,
           msg_W_1, msg_b_1, up_W_1, up_b_1, bn_g_1, bn_b_1,
           msg_W_2, msg_b_2, up_W_2, up_b_2, bn_g_2, bn_b_2,
           msg_W_3, msg_b_3, up_W_3, up_b_3, bn_g_3, bn_b_3):
    layers = [
        (msg_W_0, msg_b_0, up_W_0, up_b_0, bn_g_0, bn_b_0),
        (msg_W_1, msg_b_1, up_W_1, up_b_1, bn_g_1, bn_b_1),
        (msg_W_2, msg_b_2, up_W_2, up_b_2, bn_g_2, bn_b_2),
        (msg_W_3, msg_b_3, up_W_3, up_b_3, bn_g_3, bn_b_3),
    ]
    src = edge_index[0]
    dst = edge_index[1]
    x_pad = jnp.pad(x, ((0, 0), (0, 32 - IN_DIM)))
    w_pad = jnp.pad(in_W, ((0, 32 - IN_DIM), (0, 0)))
    ea_pad = jnp.pad(edge_attr, ((0, 0), (0, 16 - EDGE_DIM)))
    wb_all = jnp.concatenate(
        [jnp.pad(p[0][H:], ((0, 16 - EDGE_DIM), (0, 0))) for p in layers],
        axis=1)
    b_all = jnp.concatenate([p[1] for p in layers]).reshape(1, 4 * H)
    inv = 1.0 / jnp.sqrt(jnp.float32(1.0 + EPS))

    h, hw = _node_in(x_pad, w_pad, in_b.reshape(1, H),
                     jnp.pad(msg_W_0[:H], ((0, 0), (0, H))))
    eprojs = _eproj(ea_pad, wb_all, b_all)
    for i in range(DEPTH):
        _, _, uW, ub, g, b = layers[i]
        aggr4 = _edge_call(hw, src, dst, eprojs[i])
        aggr = aggr4.reshape(NC * PASSES, 2 * VROWS, H)[:, :QUART]
        aggr = aggr.reshape(NC * PASSES * QUART, H)[:N]
        wnext = layers[i + 1][0][:H] if i + 1 < DEPTH else msg_W_0[:H]
        wnext = jnp.pad(wnext, ((0, 0), (0, H)))
        h, hw = _update(aggr, h, uW[:H], uW[H:], ub.reshape(1, H),
                        (g * inv).reshape(1, H), b.reshape(1, H), wnext)
    sums2, cnt2 = _pool_call(h, batch)
    lo, hi = _combine(sums2, cnt2)
    out = jnp.stack([lo, hi], axis=1).reshape(2 * GPR, H)
    return out[:NUM_GRAPHS]


# speedup vs baseline: 1.7222x; 1.7222x over previous
import functools

import jax
import jax.numpy as jnp
from jax import lax
from jax.experimental import pallas as pl
from jax.experimental.pallas import tpu as pltpu
from jax.experimental.pallas import tpu_sc as plsc

N = 50000
E = 800000
IN_DIM = 25
EDGE_DIM = 11
H = 64
DEPTH = 4
NUM_GRAPHS = 1000
EPS = 1e-5

NC = 2
NS = 16
L = 16

HALF = 25088
VROWS = 12544
C = 64
TOTC = E // C
NITER = (TOTC + NS - 1) // NS
ZPT = VROWS // NS
ZTAIL = ZPT - (ZPT // C) * C

GPR = 512
PC = 80
NODE_CHUNKS = N // PC
PITER = (NODE_CHUNKS + NC * NS - 1) // (NC * NS)

_SC_MESH = plsc.VectorSubcoreMesh(
    core_axis_name="c", subcore_axis_name="s", num_cores=NC, num_subcores=NS)


def _leaky(t):
    return jnp.maximum(t, 0.1 * t)


def _edge_body(hw_hbm, src_hbm, dst_hbm, ep_hbm, out_hbm,
               src_v, dst_v, idx_v, rows_v, ep_v, sel_v, aggr_sh, sem):
    c = lax.axis_index("c")
    s = lax.axis_index("s")
    zerov = jnp.zeros((L,), jnp.float32)
    base_node = c * HALF

    def _zfill(i, _):
        for k in range(2 * H // L):
            sel_v[i, pl.ds(k * L, L)] = zerov
        return 0
    lax.fori_loop(0, C, _zfill, 0)
    zbase = s * ZPT
    for z in range(ZPT // C):
        pltpu.sync_copy(sel_v, aggr_sh.at[pl.ds(zbase + z * C, C)])
    if ZTAIL:
        pltpu.sync_copy(sel_v.at[pl.ds(0, ZTAIL)],
                        aggr_sh.at[pl.ds(zbase + (ZPT // C) * C, ZTAIL)])
    plsc.subcore_barrier()

    def _chunk(k, _):
        @pl.when(s + NS * k < TOTC)
        def _do():
            ebase = (s + NS * k) * C
            pltpu.sync_copy(src_hbm.at[pl.ds(ebase, C)], src_v)
            pltpu.sync_copy(dst_hbm.at[pl.ds(ebase, C)],
                            dst_v.at[pl.ds(0, C)])
            pltpu.sync_copy(ep_hbm.at[pl.ds(ebase, C)], ep_v)
            pltpu.async_copy(hw_hbm.at[src_v], rows_v, sem).wait()
            for t in range(C // L):
                d = dst_v[pl.ds(t * L, L)]
                loc = d - base_node
                ok = (loc >= 0) & (loc < HALF)
                idx_v[pl.ds(t * L, L)] = jnp.where(
                    ok, lax.shift_right_logical(loc, 1), VROWS)

            def _msg(e, _):
                d16 = dst_v[pl.ds(e, L)]
                pe = lax.rem(d16[0], 2)
                mlow = pe == 0
                mhigh = pe == 1
                for k2 in range(H // L):
                    a = rows_v[e, pl.ds(k2 * L, L)]
                    b = ep_v[e, pl.ds(k2 * L, L)]
                    v = a + b
                    m = jnp.maximum(v, 0.1 * v)
                    sel_v[e, pl.ds(k2 * L, L)] = jnp.where(mlow, m, zerov)
                    sel_v[e, pl.ds(H + k2 * L, L)] = jnp.where(mhigh, m,
                                                               zerov)
                return 0
            lax.fori_loop(0, C, _msg, 0)
            pltpu.sync_copy(sel_v, aggr_sh.at[idx_v], add=True)
        return 0
    lax.fori_loop(0, NITER, _chunk, 0)

    plsc.subcore_barrier()
    pltpu.sync_copy(aggr_sh.at[pl.ds(s * ZPT, ZPT)],
                    out_hbm.at[c, pl.ds(s * ZPT, ZPT)])
    plsc.subcore_barrier()


_edge_call = pl.kernel(
    _edge_body,
    out_type=jax.ShapeDtypeStruct((NC, VROWS, 2 * H), jnp.float32),
    mesh=_SC_MESH,
    scratch_types=[
        pltpu.VMEM((C,), jnp.int32),
        pltpu.VMEM((C + L,), jnp.int32),
        pltpu.VMEM((C,), jnp.int32),
        pltpu.VMEM((C, 2 * H), jnp.float32),
        pltpu.VMEM((C, H), jnp.float32),
        pltpu.VMEM((C, 2 * H), jnp.float32),
        pltpu.VMEM_SHARED((VROWS + 8, 2 * H), jnp.float32),
        pltpu.SemaphoreType.DMA,
    ],
)


def _pool_body(h_hbm, batch_hbm, sums_hbm, cnt_hbm,
               bidx_v, idx_v, h_v, hsel_v, csel_v, sums_sh, cnt_sh, sem):
    c = lax.axis_index("c")
    s = lax.axis_index("s")
    wid = s * NC + c
    zerov = jnp.zeros((L,), jnp.float32)
    lane = lax.broadcasted_iota(jnp.int32, (L,), 0)
    onerow = jnp.where(lane == 0, 1.0, 0.0).astype(jnp.float32)

    def _zf(i, _):
        for k in range(2 * H // L):
            hsel_v[i, pl.ds(k * L, L)] = zerov
            csel_v[i, pl.ds(k * L, L)] = zerov
        return 0
    lax.fori_loop(0, PC, _zf, 0)

    zpt = GPR // NS
    pltpu.sync_copy(hsel_v.at[pl.ds(0, zpt)], sums_sh.at[pl.ds(s * zpt, zpt)])
    pltpu.sync_copy(csel_v.at[pl.ds(0, zpt)], cnt_sh.at[pl.ds(s * zpt, zpt)])
    plsc.subcore_barrier()

    for i in range(PITER):
        k = wid + NC * NS * i

        @pl.when(k < NODE_CHUNKS)
        def _do(k=k):
            nbase = k * PC
            pltpu.sync_copy(batch_hbm.at[pl.ds(nbase, PC)],
                            bidx_v.at[pl.ds(0, PC)])
            pltpu.sync_copy(h_hbm.at[pl.ds(nbase, PC)], h_v)
            for t in range(PC // L):
                g = bidx_v[pl.ds(t * L, L)]
                idx_v[pl.ds(t * L, L)] = lax.shift_right_logical(g, 1)

            def _sel(e, _):
                g16 = bidx_v[pl.ds(e, L)]
                pe = lax.rem(g16[0], 2)
                for k2 in range(H // L):
                    m = h_v[e, pl.ds(k2 * L, L)]
                    hsel_v[e, pl.ds(k2 * L, L)] = jnp.where(pe == 0, m, zerov)
                    hsel_v[e, pl.ds(H + k2 * L, L)] = jnp.where(pe == 1, m,
                                                                zerov)
                csel_v[e, pl.ds(0, L)] = jnp.where(pe == 0, onerow, zerov)
                csel_v[e, pl.ds(H, L)] = jnp.where(pe == 1, onerow, zerov)
                return 0
            lax.fori_loop(0, PC, _sel, 0)
            pltpu.sync_copy(hsel_v, sums_sh.at[idx_v], add=True)
            pltpu.sync_copy(csel_v, cnt_sh.at[idx_v], add=True)

    plsc.subcore_barrier()
    pltpu.sync_copy(sums_sh.at[pl.ds(s * zpt, zpt)],
                    sums_hbm.at[c, pl.ds(s * zpt, zpt)])
    pltpu.sync_copy(cnt_sh.at[pl.ds(s * zpt, zpt)],
                    cnt_hbm.at[c, pl.ds(s * zpt, zpt)])


_pool_call = pl.kernel(
    _pool_body,
    out_type=(jax.ShapeDtypeStruct((NC, GPR, 2 * H), jnp.float32),
              jax.ShapeDtypeStruct((NC, GPR, 2 * H), jnp.float32)),
    mesh=_SC_MESH,
    scratch_types=[
        pltpu.VMEM((PC + L,), jnp.int32),
        pltpu.VMEM((PC,), jnp.int32),
        pltpu.VMEM((PC, H), jnp.float32),
        pltpu.VMEM((PC, 2 * H), jnp.float32),
        pltpu.VMEM((PC, 2 * H), jnp.float32),
        pltpu.VMEM_SHARED((GPR, 2 * H), jnp.float32),
        pltpu.VMEM_SHARED((GPR, 2 * H), jnp.float32),
        pltpu.SemaphoreType.DMA,
    ],
)


BN = 2000
BE = 4000


def _node_in_body(x_ref, w_ref, b_ref, wt_ref, h_ref, hw_ref):
    h = _leaky(jnp.dot(x_ref[...], w_ref[...],
                       preferred_element_type=jnp.float32) + b_ref[...])
    h_ref[...] = h
    hw_ref[...] = jnp.dot(h, wt_ref[...], preferred_element_type=jnp.float32)


def _node_in(x_pad, w_pad, b, wt):
    return pl.pallas_call(
        _node_in_body,
        grid=(N // BN,),
        in_specs=[pl.BlockSpec((BN, 32), lambda i: (i, 0)),
                  pl.BlockSpec((32, H), lambda i: (0, 0)),
                  pl.BlockSpec((1, H), lambda i: (0, 0)),
                  pl.BlockSpec((H, 2 * H), lambda i: (0, 0))],
        out_specs=[pl.BlockSpec((BN, H), lambda i: (i, 0)),
                   pl.BlockSpec((BN, 2 * H), lambda i: (i, 0))],
        out_shape=[jax.ShapeDtypeStruct((N, H), jnp.float32),
                   jax.ShapeDtypeStruct((N, 2 * H), jnp.float32)],
    )(x_pad, w_pad, b, wt)


def _eproj_body(ea_ref, wb_ref, bias_ref, o0, o1, o2, o3):
    t = jnp.dot(ea_ref[...], wb_ref[...],
                preferred_element_type=jnp.float32) + bias_ref[...]
    o0[...] = t[:, 0:H]
    o1[...] = t[:, H:2 * H]
    o2[...] = t[:, 2 * H:3 * H]
    o3[...] = t[:, 3 * H:4 * H]


def _eproj(ea_pad, wb_all, b_all):
    return pl.pallas_call(
        _eproj_body,
        grid=(E // BE,),
        in_specs=[pl.BlockSpec((BE, 16), lambda i: (i, 0)),
                  pl.BlockSpec((16, 4 * H), lambda i: (0, 0)),
                  pl.BlockSpec((1, 4 * H), lambda i: (0, 0))],
        out_specs=[pl.BlockSpec((BE, H), lambda i: (i, 0))] * 4,
        out_shape=[jax.ShapeDtypeStruct((E, H), jnp.float32)] * 4,
    )(ea_pad, wb_all, b_all)


def _update_body(a_ref, h_ref, wua_ref, wuh_ref, ub_ref, g_ref, bb_ref,
                 wn_ref, hn_ref, hwn_ref):
    z = (jnp.dot(a_ref[...], wua_ref[...], preferred_element_type=jnp.float32)
         + jnp.dot(h_ref[...], wuh_ref[...], preferred_element_type=jnp.float32)
         + ub_ref[...])
    r = jnp.maximum(z, 0.0)
    hn = g_ref[...] * r + bb_ref[...]
    hn_ref[...] = hn
    hwn_ref[...] = jnp.dot(hn, wn_ref[...], preferred_element_type=jnp.float32)


def _update(aggr, h, wua, wuh, ub, g, bb, wn):
    return pl.pallas_call(
        _update_body,
        grid=(N // BN,),
        in_specs=[pl.BlockSpec((BN, H), lambda i: (i, 0)),
                  pl.BlockSpec((BN, H), lambda i: (i, 0)),
                  pl.BlockSpec((H, H), lambda i: (0, 0)),
                  pl.BlockSpec((H, H), lambda i: (0, 0)),
                  pl.BlockSpec((1, H), lambda i: (0, 0)),
                  pl.BlockSpec((1, H), lambda i: (0, 0)),
                  pl.BlockSpec((1, H), lambda i: (0, 0)),
                  pl.BlockSpec((H, 2 * H), lambda i: (0, 0))],
        out_specs=[pl.BlockSpec((BN, H), lambda i: (i, 0)),
                   pl.BlockSpec((BN, 2 * H), lambda i: (i, 0))],
        out_shape=[jax.ShapeDtypeStruct((N, H), jnp.float32),
                   jax.ShapeDtypeStruct((N, 2 * H), jnp.float32)],
    )(aggr, h, wua, wuh, ub, g, bb, wn)


def _combine_body(s_ref, c_ref, lo_ref, hi_ref):
    sm = s_ref[0] + s_ref[1]
    ct = c_ref[0] + c_ref[1]
    lo_ref[...] = sm[:, 0:H] / jnp.maximum(ct[:, 0:1], 1.0)
    hi_ref[...] = sm[:, H:2 * H] / jnp.maximum(ct[:, H:H + 1], 1.0)


def _combine(sums2, cnt2):
    return pl.pallas_call(
        _combine_body,
        out_shape=[jax.ShapeDtypeStruct((GPR, H), jnp.float32),
                   jax.ShapeDtypeStruct((GPR, H), jnp.float32)],
    )(sums2, cnt2)


def kernel(x, edge_index, edge_attr, batch, in_W, in_b,
           msg_W_0, msg_b_0, up_W_0, up_b_0, bn_g_0, bn_b_0,
           msg_W_1, msg_b_1, up_W_1, up_b_1, bn_g_1, bn_b_1,
           msg_W_2, msg_b_2, up_W_2, up_b_2, bn_g_2, bn_b_2,
           msg_W_3, msg_b_3, up_W_3, up_b_3, bn_g_3, bn_b_3):
    layers = [
        (msg_W_0, msg_b_0, up_W_0, up_b_0, bn_g_0, bn_b_0),
        (msg_W_1, msg_b_1, up_W_1, up_b_1, bn_g_1, bn_b_1),
        (msg_W_2, msg_b_2, up_W_2, up_b_2, bn_g_2, bn_b_2),
        (msg_W_3, msg_b_3, up_W_3, up_b_3, bn_g_3, bn_b_3),
    ]
    src = edge_index[0]
    dst = edge_index[1]
    x_pad = jnp.pad(x, ((0, 0), (0, 32 - IN_DIM)))
    w_pad = jnp.pad(in_W, ((0, 32 - IN_DIM), (0, 0)))
    ea_pad = jnp.pad(edge_attr, ((0, 0), (0, 16 - EDGE_DIM)))
    wb_all = jnp.concatenate(
        [jnp.pad(p[0][H:], ((0, 16 - EDGE_DIM), (0, 0))) for p in layers],
        axis=1)
    b_all = jnp.concatenate([p[1] for p in layers]).reshape(1, 4 * H)
    inv = 1.0 / jnp.sqrt(jnp.float32(1.0 + EPS))

    h, hw = _node_in(x_pad, w_pad, in_b.reshape(1, H),
                     jnp.pad(msg_W_0[:H], ((0, 0), (0, H))))
    eprojs = _eproj(ea_pad, wb_all, b_all)
    for i in range(DEPTH):
        _, _, uW, ub, g, b = layers[i]
        aggr4 = _edge_call(hw, src, dst, eprojs[i])
        aggr = aggr4.reshape(NC * 2 * VROWS, H)[:N]
        wnext = layers[i + 1][0][:H] if i + 1 < DEPTH else msg_W_0[:H]
        wnext = jnp.pad(wnext, ((0, 0), (0, H)))
        h, hw = _update(aggr, h, uW[:H], uW[H:], ub.reshape(1, H),
                        (g * inv).reshape(1, H), b.reshape(1, H), wnext)
    sums2, cnt2 = _pool_call(h, batch)
    lo, hi = _combine(sums2, cnt2)
    out = jnp.stack([lo, hi], axis=1).reshape(2 * GPR, H)
    return out[:NUM_GRAPHS]


# speedup vs baseline: 1.8936x; 1.0995x over previous
import functools

import jax
import jax.numpy as jnp
from jax import lax
from jax.experimental import pallas as pl
from jax.experimental.pallas import tpu as pltpu
from jax.experimental.pallas import tpu_sc as plsc

N = 50000
E = 800000
IN_DIM = 25
EDGE_DIM = 11
H = 64
DEPTH = 4
NUM_GRAPHS = 1000
EPS = 1e-5

NC = 2
NS = 16
L = 16

HALF = 25088
VROWS = 12544
C = 80
TOTC = E // C
NITER = (TOTC + NS - 1) // NS
ZPT = VROWS // NS
ZTAIL = ZPT - (ZPT // C) * C

GPR = 512
PC = 80
NODE_CHUNKS = N // PC
PITER = (NODE_CHUNKS + NC * NS - 1) // (NC * NS)

_SC_MESH = plsc.VectorSubcoreMesh(
    core_axis_name="c", subcore_axis_name="s", num_cores=NC, num_subcores=NS)


def _leaky(t):
    return jnp.maximum(t, 0.1 * t)


def _edge_body(hw_hbm, src_hbm, dst_hbm, ep_hbm, out_hbm,
               src_v, dst_v, idx_v, rows_v, ep_v, aggr_sh, sem):
    c = lax.axis_index("c")
    s = lax.axis_index("s")
    zerov = jnp.zeros((L,), jnp.float32)
    base_node = c * HALF

    def _zfill(i, _):
        for k in range(2 * H // L):
            rows_v[i, pl.ds(k * L, L)] = zerov
        return 0
    lax.fori_loop(0, C, _zfill, 0)
    zbase = s * ZPT
    for z in range(ZPT // C):
        pltpu.sync_copy(rows_v, aggr_sh.at[pl.ds(zbase + z * C, C)])
    if ZTAIL:
        pltpu.sync_copy(rows_v.at[pl.ds(0, ZTAIL)],
                        aggr_sh.at[pl.ds(zbase + (ZPT // C) * C, ZTAIL)])
    plsc.subcore_barrier()

    def _chunk(k, _):
        @pl.when(s + NS * k < TOTC)
        def _do():
            ebase = (s + NS * k) * C
            pltpu.sync_copy(src_hbm.at[pl.ds(ebase, C)], src_v)
            pltpu.sync_copy(dst_hbm.at[pl.ds(ebase, C)],
                            dst_v.at[pl.ds(0, C)])
            pltpu.sync_copy(ep_hbm.at[pl.ds(ebase, C)], ep_v)
            pltpu.async_copy(hw_hbm.at[src_v], rows_v, sem).wait()
            for t in range(C // L):
                d = dst_v[pl.ds(t * L, L)]
                loc = d - base_node
                ok = (loc >= 0) & (loc < HALF)
                idx_v[pl.ds(t * L, L)] = jnp.where(
                    ok, lax.shift_right_logical(loc, 1), VROWS)

            def _msg(e, _):
                d16 = dst_v[pl.ds(e, L)]
                pe = lax.rem(d16[0], 2)
                mlow = pe == 0
                mhigh = pe == 1
                for k2 in range(H // L):
                    a = rows_v[e, pl.ds(k2 * L, L)]
                    b = ep_v[e, pl.ds(k2 * L, L)]
                    v = a + b
                    m = jnp.maximum(v, 0.1 * v)
                    rows_v[e, pl.ds(k2 * L, L)] = jnp.where(mlow, m, zerov)
                    rows_v[e, pl.ds(H + k2 * L, L)] = jnp.where(mhigh, m,
                                                                zerov)
                return 0
            lax.fori_loop(0, C, _msg, 0)
            pltpu.sync_copy(rows_v, aggr_sh.at[idx_v], add=True)
        return 0
    lax.fori_loop(0, NITER, _chunk, 0)

    plsc.subcore_barrier()
    pltpu.sync_copy(aggr_sh.at[pl.ds(s * ZPT, ZPT)],
                    out_hbm.at[c, pl.ds(s * ZPT, ZPT)])
    plsc.subcore_barrier()


_edge_call = pl.kernel(
    _edge_body,
    out_type=jax.ShapeDtypeStruct((NC, VROWS, 2 * H), jnp.float32),
    mesh=_SC_MESH,
    scratch_types=[
        pltpu.VMEM((C,), jnp.int32),
        pltpu.VMEM((C + L,), jnp.int32),
        pltpu.VMEM((C,), jnp.int32),
        pltpu.VMEM((C, 2 * H), jnp.float32),
        pltpu.VMEM((C, H), jnp.float32),
        pltpu.VMEM_SHARED((VROWS + 8, 2 * H), jnp.float32),
        pltpu.SemaphoreType.DMA,
    ],
)


def _pool_body(h_hbm, batch_hbm, sums_hbm, cnt_hbm,
               bidx_v, idx_v, h_v, hsel_v, csel_v, sums_sh, cnt_sh, sem):
    c = lax.axis_index("c")
    s = lax.axis_index("s")
    wid = s * NC + c
    zerov = jnp.zeros((L,), jnp.float32)
    lane = lax.broadcasted_iota(jnp.int32, (L,), 0)
    onerow = jnp.where(lane == 0, 1.0, 0.0).astype(jnp.float32)

    def _zf(i, _):
        for k in range(2 * H // L):
            hsel_v[i, pl.ds(k * L, L)] = zerov
            csel_v[i, pl.ds(k * L, L)] = zerov
        return 0
    lax.fori_loop(0, PC, _zf, 0)

    zpt = GPR // NS
    pltpu.sync_copy(hsel_v.at[pl.ds(0, zpt)], sums_sh.at[pl.ds(s * zpt, zpt)])
    pltpu.sync_copy(csel_v.at[pl.ds(0, zpt)], cnt_sh.at[pl.ds(s * zpt, zpt)])
    plsc.subcore_barrier()

    for i in range(PITER):
        k = wid + NC * NS * i

        @pl.when(k < NODE_CHUNKS)
        def _do(k=k):
            nbase = k * PC
            pltpu.sync_copy(batch_hbm.at[pl.ds(nbase, PC)],
                            bidx_v.at[pl.ds(0, PC)])
            pltpu.sync_copy(h_hbm.at[pl.ds(nbase, PC)], h_v)
            for t in range(PC // L):
                g = bidx_v[pl.ds(t * L, L)]
                idx_v[pl.ds(t * L, L)] = lax.shift_right_logical(g, 1)

            def _sel(e, _):
                g16 = bidx_v[pl.ds(e, L)]
                pe = lax.rem(g16[0], 2)
                for k2 in range(H // L):
                    m = h_v[e, pl.ds(k2 * L, L)]
                    hsel_v[e, pl.ds(k2 * L, L)] = jnp.where(pe == 0, m, zerov)
                    hsel_v[e, pl.ds(H + k2 * L, L)] = jnp.where(pe == 1, m,
                                                                zerov)
                csel_v[e, pl.ds(0, L)] = jnp.where(pe == 0, onerow, zerov)
                csel_v[e, pl.ds(H, L)] = jnp.where(pe == 1, onerow, zerov)
                return 0
            lax.fori_loop(0, PC, _sel, 0)
            pltpu.sync_copy(hsel_v, sums_sh.at[idx_v], add=True)
            pltpu.sync_copy(csel_v, cnt_sh.at[idx_v], add=True)

    plsc.subcore_barrier()
    pltpu.sync_copy(sums_sh.at[pl.ds(s * zpt, zpt)],
                    sums_hbm.at[c, pl.ds(s * zpt, zpt)])
    pltpu.sync_copy(cnt_sh.at[pl.ds(s * zpt, zpt)],
                    cnt_hbm.at[c, pl.ds(s * zpt, zpt)])


_pool_call = pl.kernel(
    _pool_body,
    out_type=(jax.ShapeDtypeStruct((NC, GPR, 2 * H), jnp.float32),
              jax.ShapeDtypeStruct((NC, GPR, 2 * H), jnp.float32)),
    mesh=_SC_MESH,
    scratch_types=[
        pltpu.VMEM((PC + L,), jnp.int32),
        pltpu.VMEM((PC,), jnp.int32),
        pltpu.VMEM((PC, H), jnp.float32),
        pltpu.VMEM((PC, 2 * H), jnp.float32),
        pltpu.VMEM((PC, 2 * H), jnp.float32),
        pltpu.VMEM_SHARED((GPR, 2 * H), jnp.float32),
        pltpu.VMEM_SHARED((GPR, 2 * H), jnp.float32),
        pltpu.SemaphoreType.DMA,
    ],
)


BN = 2000
BE = 4000


def _node_in_body(x_ref, w_ref, b_ref, wt_ref, h_ref, hw_ref):
    h = _leaky(jnp.dot(x_ref[...], w_ref[...],
                       preferred_element_type=jnp.float32) + b_ref[...])
    h_ref[...] = h
    hw_ref[...] = jnp.dot(h, wt_ref[...], preferred_element_type=jnp.float32)


def _node_in(x_pad, w_pad, b, wt):
    return pl.pallas_call(
        _node_in_body,
        grid=(N // BN,),
        in_specs=[pl.BlockSpec((BN, 32), lambda i: (i, 0)),
                  pl.BlockSpec((32, H), lambda i: (0, 0)),
                  pl.BlockSpec((1, H), lambda i: (0, 0)),
                  pl.BlockSpec((H, 2 * H), lambda i: (0, 0))],
        out_specs=[pl.BlockSpec((BN, H), lambda i: (i, 0)),
                   pl.BlockSpec((BN, 2 * H), lambda i: (i, 0))],
        out_shape=[jax.ShapeDtypeStruct((N, H), jnp.float32),
                   jax.ShapeDtypeStruct((N, 2 * H), jnp.float32)],
    )(x_pad, w_pad, b, wt)


def _eproj_body(ea_ref, wb_ref, bias_ref, o0, o1, o2, o3):
    t = jnp.dot(ea_ref[...], wb_ref[...],
                preferred_element_type=jnp.float32) + bias_ref[...]
    o0[...] = t[:, 0:H]
    o1[...] = t[:, H:2 * H]
    o2[...] = t[:, 2 * H:3 * H]
    o3[...] = t[:, 3 * H:4 * H]


def _eproj(ea_pad, wb_all, b_all):
    return pl.pallas_call(
        _eproj_body,
        grid=(E // BE,),
        in_specs=[pl.BlockSpec((BE, 16), lambda i: (i, 0)),
                  pl.BlockSpec((16, 4 * H), lambda i: (0, 0)),
                  pl.BlockSpec((1, 4 * H), lambda i: (0, 0))],
        out_specs=[pl.BlockSpec((BE, H), lambda i: (i, 0))] * 4,
        out_shape=[jax.ShapeDtypeStruct((E, H), jnp.float32)] * 4,
    )(ea_pad, wb_all, b_all)


def _update_body(a_ref, h_ref, wua_ref, wuh_ref, ub_ref, g_ref, bb_ref,
                 wn_ref, hn_ref, hwn_ref):
    z = (jnp.dot(a_ref[...], wua_ref[...], preferred_element_type=jnp.float32)
         + jnp.dot(h_ref[...], wuh_ref[...], preferred_element_type=jnp.float32)
         + ub_ref[...])
    r = jnp.maximum(z, 0.0)
    hn = g_ref[...] * r + bb_ref[...]
    hn_ref[...] = hn
    hwn_ref[...] = jnp.dot(hn, wn_ref[...], preferred_element_type=jnp.float32)


def _update(aggr, h, wua, wuh, ub, g, bb, wn):
    return pl.pallas_call(
        _update_body,
        grid=(N // BN,),
        in_specs=[pl.BlockSpec((BN, H), lambda i: (i, 0)),
                  pl.BlockSpec((BN, H), lambda i: (i, 0)),
                  pl.BlockSpec((H, H), lambda i: (0, 0)),
                  pl.BlockSpec((H, H), lambda i: (0, 0)),
                  pl.BlockSpec((1, H), lambda i: (0, 0)),
                  pl.BlockSpec((1, H), lambda i: (0, 0)),
                  pl.BlockSpec((1, H), lambda i: (0, 0)),
                  pl.BlockSpec((H, 2 * H), lambda i: (0, 0))],
        out_specs=[pl.BlockSpec((BN, H), lambda i: (i, 0)),
                   pl.BlockSpec((BN, 2 * H), lambda i: (i, 0))],
        out_shape=[jax.ShapeDtypeStruct((N, H), jnp.float32),
                   jax.ShapeDtypeStruct((N, 2 * H), jnp.float32)],
    )(aggr, h, wua, wuh, ub, g, bb, wn)


def _combine_body(s_ref, c_ref, lo_ref, hi_ref):
    sm = s_ref[0] + s_ref[1]
    ct = c_ref[0] + c_ref[1]
    lo_ref[...] = sm[:, 0:H] / jnp.maximum(ct[:, 0:1], 1.0)
    hi_ref[...] = sm[:, H:2 * H] / jnp.maximum(ct[:, H:H + 1], 1.0)


def _combine(sums2, cnt2):
    return pl.pallas_call(
        _combine_body,
        out_shape=[jax.ShapeDtypeStruct((GPR, H), jnp.float32),
                   jax.ShapeDtypeStruct((GPR, H), jnp.float32)],
    )(sums2, cnt2)


def kernel(x, edge_index, edge_attr, batch, in_W, in_b,
           msg_W_0, msg_b_0, up_W_0, up_b_0, bn_g_0, bn_b_0,
           msg_W_1, msg_b_1, up_W_1, up_b_1, bn_g_1, bn_b_1,
           msg_W_2, msg_b_2, up_W_2, up_b_2, bn_g_2, bn_b_2,
           msg_W_3, msg_b_3, up_W_3, up_b_3, bn_g_3, bn_b_3):
    layers = [
        (msg_W_0, msg_b_0, up_W_0, up_b_0, bn_g_0, bn_b_0),
        (msg_W_1, msg_b_1, up_W_1, up_b_1, bn_g_1, bn_b_1),
        (msg_W_2, msg_b_2, up_W_2, up_b_2, bn_g_2, bn_b_2),
        (msg_W_3, msg_b_3, up_W_3, up_b_3, bn_g_3, bn_b_3),
    ]
    src = edge_index[0]
    dst = edge_index[1]
    x_pad = jnp.pad(x, ((0, 0), (0, 32 - IN_DIM)))
    w_pad = jnp.pad(in_W, ((0, 32 - IN_DIM), (0, 0)))
    ea_pad = jnp.pad(edge_attr, ((0, 0), (0, 16 - EDGE_DIM)))
    wb_all = jnp.concatenate(
        [jnp.pad(p[0][H:], ((0, 16 - EDGE_DIM), (0, 0))) for p in layers],
        axis=1)
    b_all = jnp.concatenate([p[1] for p in layers]).reshape(1, 4 * H)
    inv = 1.0 / jnp.sqrt(jnp.float32(1.0 + EPS))

    h, hw = _node_in(x_pad, w_pad, in_b.reshape(1, H),
                     jnp.pad(msg_W_0[:H], ((0, 0), (0, H))))
    eprojs = _eproj(ea_pad, wb_all, b_all)
    for i in range(DEPTH):
        _, _, uW, ub, g, b = layers[i]
        aggr4 = _edge_call(hw, src, dst, eprojs[i])
        aggr = aggr4.reshape(NC * 2 * VROWS, H)[:N]
        wnext = layers[i + 1][0][:H] if i + 1 < DEPTH else msg_W_0[:H]
        wnext = jnp.pad(wnext, ((0, 0), (0, H)))
        h, hw = _update(aggr, h, uW[:H], uW[H:], ub.reshape(1, H),
                        (g * inv).reshape(1, H), b.reshape(1, H), wnext)
    sums2, cnt2 = _pool_call(h, batch)
    lo, hi = _combine(sums2, cnt2)
    out = jnp.stack([lo, hi], axis=1).reshape(2 * GPR, H)
    return out[:NUM_GRAPHS]
